# baseline (speedup 1.0000x reference)
import numpy as np

import jax
import jax.numpy as jnp
from jax import lax
from jax.experimental import pallas as pl
from jax.experimental.pallas import tpu as pltpu
from jax.experimental.pallas import tpu_sc as plsc

B = 4
NPOINT = 1024
N = 16384
K = 32
L = 16
NV = N // L
NTEC = 32
CPT = (B * NPOINT) // NTEC
TPB = NTEC // B
CAP = N + 2 * L
NQ = NV // 4

F32_INF = np.float32(np.inf)
F32_NINF = np.float32(-np.inf)
I32_MAX = np.int32(2**31 - 1)


def _rev(x):
    return lax.rev(x, (0,))


def _lexminmax(ak, av, bk, bv):
    m = (ak < bk) | ((ak == bk) & (av < bv))
    return (jnp.where(m, ak, bk), jnp.where(m, av, bv),
            jnp.where(m, bk, ak), jnp.where(m, bv, av))


def _knn_body(xt, ct, out, xv, yv, zv, cxv, cyv, czv, distv, cand_i,
              minv, wl, outbuf):
    wid = lax.axis_index("s") * 2 + lax.axis_index("c")
    b = wid // TPB
    c0 = (wid % TPB) * CPT

    pltpu.sync_copy(xt.at[pl.ds((b * 3 + 0) * N, N)], xv)
    pltpu.sync_copy(xt.at[pl.ds((b * 3 + 1) * N, N)], yv)
    pltpu.sync_copy(xt.at[pl.ds((b * 3 + 2) * N, N)], zv)
    pltpu.sync_copy(ct.at[pl.ds(((b * 3 + 0) * NPOINT + c0) * L, CPT * L)], cxv)
    pltpu.sync_copy(ct.at[pl.ds(((b * 3 + 1) * NPOINT + c0) * L, CPT * L)], cyv)
    pltpu.sync_copy(ct.at[pl.ds(((b * 3 + 2) * NPOINT + c0) * L, CPT * L)], czv)

    iota = lax.iota(jnp.int32, L)
    inf16 = jnp.full((L,), F32_INF, jnp.float32)
    imax16 = jnp.full((L,), I32_MAX, jnp.int32)
    lane15 = iota == (L - 1)
    izero = jnp.zeros((L,), jnp.int32)
    for r in range(4):
        distv[pl.ds(N + r * L, L)] = inf16

    def center_body(ci, _):
        cx = cxv[pl.ds(ci * L, L)]
        cy = cyv[pl.ds(ci * L, L)]
        cz = czv[pl.ds(ci * L, L)]

        @plsc.parallel_loop(0, NV, step=4, unroll=2,
                            carry=(inf16, inf16, inf16, inf16))
        def p1_loop(j, accs):
            new = []
            ds = []
            for r, a in enumerate(accs):
                off = (j + r) * L
                x = xv[pl.ds(off, L)]
                y = yv[pl.ds(off, L)]
                z = zv[pl.ds(off, L)]
                dx = cx - x
                dy = cy - y
                dz = cz - z
                d = (dx * dx + dy * dy) + dz * dz
                distv[pl.ds(off, L)] = d
                ds.append(d)
                new.append(jnp.minimum(a, d))
            mq = jnp.minimum(jnp.minimum(ds[0], ds[1]),
                             jnp.minimum(ds[2], ds[3]))
            cm = plsc.cummax(jnp.float32(0) - mq)
            plsc.store_scatter(minv, [izero + j // 4], cm, mask=lane15)
            return tuple(new)

        a0, a1, a2, a3 = p1_loop

        s0, s1 = jnp.sort(a0), jnp.sort(a1)
        s2, s3 = jnp.sort(a2), jnp.sort(a3)
        r1, r3 = _rev(s1), _rev(s3)
        p0 = jnp.sort(jnp.minimum(s0, r1))
        p1 = jnp.sort(jnp.maximum(s0, r1))
        q0 = jnp.sort(jnp.minimum(s2, r3))
        q1 = jnp.sort(jnp.maximum(s2, r3))
        lo0 = jnp.minimum(p0, _rev(q1))
        lo1 = jnp.minimum(p1, _rev(q0))
        t = jnp.max(jnp.maximum(lo0, lo1))

        tneg = jnp.float32(0) - t

        @plsc.parallel_loop(0, NQ // L, unroll=2, carry=jnp.int32(0))
        def p2a_loop(g, nw):
            mv = minv[pl.ds(g * L, L)]
            sel = mv >= tneg
            cnt = plsc.all_reduce_population_count(sel)[0]
            plsc.store_compressed(wl.at[pl.ds(nw, L)], iota + g * L, mask=sel)
            return nw + cnt

        nw = p2a_loop
        plsc.store_compressed(wl.at[pl.ds(nw, L)], izero + NQ,
                              mask=iota >= 0)
        ng = (nw + L - 1) // L

        def p2b_group(g, off):
            wlv = wl[pl.ds(g * L, L)]
            for k in range(L):
                base = wlv[k] * (4 * L)
                for r in range(4):
                    o16 = base + r * L
                    d = distv[pl.ds(o16, L)]
                    sel = d <= t
                    cnt = plsc.all_reduce_population_count(sel)[0]
                    plsc.store_compressed(cand_i.at[pl.ds(off, L)],
                                          iota + o16, mask=sel)
                    off = off + cnt
            return off

        c = lax.fori_loop(0, ng, p2b_group, jnp.int32(0))
        nv = (c + L - 1) // L

        def scan_body(v, st):
            a0k, a0v, a1k, a1v = st
            o16 = v * L
            ii = cand_i[pl.ds(o16, L)]
            valid = (iota + o16) < c
            d = plsc.load_gather(distv, [ii], mask=valid)
            d = jnp.where(valid, d, F32_INF)
            ii = jnp.where(valid, ii, I32_MAX)
            sk, sv = plsc.sort_key_val(d, ii)
            mk, mv, _, _ = _lexminmax(a1k, a1v, _rev(sk), _rev(sv))
            mk, mv = plsc.sort_key_val(mk, mv)
            lok, lov, hik, hiv = _lexminmax(a0k, a0v, _rev(mk), _rev(mv))
            a0k, a0v = plsc.sort_key_val(lok, lov)
            a1k, a1v = plsc.sort_key_val(hik, hiv)
            return (a0k, a0v, a1k, a1v)

        _, ov0, _, ov1 = lax.fori_loop(0, nv, scan_body,
                                       (inf16, imax16, inf16, imax16))
        outbuf[pl.ds(ci * K, L)] = ov0
        outbuf[pl.ds(ci * K + L, L)] = ov1
        return 0

    lax.fori_loop(0, CPT, center_body, 0)
    pltpu.sync_copy(outbuf, out.at[pl.ds(wid * (CPT * K), CPT * K)])


@jax.jit
def _knn(xt, ct):
    f = pl.kernel(
        _knn_body,
        out_type=jax.ShapeDtypeStruct((B * NPOINT * K,), jnp.int32),
        mesh=plsc.VectorSubcoreMesh(core_axis_name="c", subcore_axis_name="s"),
        compiler_params=pltpu.CompilerParams(needs_layout_passes=False),
        scratch_types=[
            pltpu.VMEM((N,), jnp.float32),
            pltpu.VMEM((N,), jnp.float32),
            pltpu.VMEM((N,), jnp.float32),
            pltpu.VMEM((CPT * L,), jnp.float32),
            pltpu.VMEM((CPT * L,), jnp.float32),
            pltpu.VMEM((CPT * L,), jnp.float32),
            pltpu.VMEM((N + 4 * L,), jnp.float32),
            pltpu.VMEM((CAP,), jnp.int32),
            pltpu.VMEM((NQ,), jnp.float32),
            pltpu.VMEM((NQ + 2 * L,), jnp.int32),
            pltpu.VMEM((CPT * K,), jnp.int32),
        ],
    )
    return f(xt, ct)


def kernel(xyz, center):
    xt = jnp.transpose(xyz, (0, 2, 1)).reshape(B * 3 * N)
    ct = jnp.repeat(jnp.transpose(center, (0, 2, 1)).reshape(B * 3 * NPOINT), L)
    return _knn(xt, ct).reshape(B, NPOINT, K)


# speedup vs baseline: 1.2247x; 1.2247x over previous
import numpy as np

import jax
import jax.numpy as jnp
from jax import lax
from jax.experimental import pallas as pl
from jax.experimental.pallas import tpu as pltpu
from jax.experimental.pallas import tpu_sc as plsc

B = 4
NPOINT = 1024
N = 16384
K = 32
L = 16
NV = N // L
NTEC = 32
CPT = (B * NPOINT) // NTEC
TPB = NTEC // B
CAP = N + 2 * L

F32_INF = np.float32(np.inf)
F32_NINF = np.float32(-np.inf)
I32_MAX = np.int32(2**31 - 1)


def _rev(x):
    return lax.rev(x, (0,))


def _lexminmax(ak, av, bk, bv):
    m = (ak < bk) | ((ak == bk) & (av < bv))
    return (jnp.where(m, ak, bk), jnp.where(m, av, bv),
            jnp.where(m, bk, ak), jnp.where(m, bv, av))


def _knn_body(xt, ct, out, xv, yv, zv, cxv, cyv, czv, distv, cand_a, cand_b,
              outbuf):
    wid = lax.axis_index("s") * 2 + lax.axis_index("c")
    b = wid // TPB
    c0 = (wid % TPB) * CPT

    pltpu.sync_copy(xt.at[pl.ds((b * 3 + 0) * N, N)], xv)
    pltpu.sync_copy(xt.at[pl.ds((b * 3 + 1) * N, N)], yv)
    pltpu.sync_copy(xt.at[pl.ds((b * 3 + 2) * N, N)], zv)
    pltpu.sync_copy(ct.at[pl.ds(((b * 3 + 0) * NPOINT + c0) * L, CPT * L)], cxv)
    pltpu.sync_copy(ct.at[pl.ds(((b * 3 + 1) * NPOINT + c0) * L, CPT * L)], cyv)
    pltpu.sync_copy(ct.at[pl.ds(((b * 3 + 2) * NPOINT + c0) * L, CPT * L)], czv)

    iota = lax.iota(jnp.int32, L)
    inf16 = jnp.full((L,), F32_INF, jnp.float32)
    imax16 = jnp.full((L,), I32_MAX, jnp.int32)

    def center_body(ci, _):
        cx = cxv[pl.ds(ci * L, L)]
        cy = cyv[pl.ds(ci * L, L)]
        cz = czv[pl.ds(ci * L, L)]

        @plsc.parallel_loop(0, NV, step=4, unroll=2,
                            carry=(inf16, inf16, inf16, inf16))
        def p1_loop(j, accs):
            new = []
            for r, a in enumerate(accs):
                off = (j + r) * L
                x = xv[pl.ds(off, L)]
                y = yv[pl.ds(off, L)]
                z = zv[pl.ds(off, L)]
                dx = cx - x
                dy = cy - y
                dz = cz - z
                d = (dx * dx + dy * dy) + dz * dz
                distv[pl.ds(off, L)] = d
                new.append(jnp.minimum(a, d))
            return tuple(new)

        a0, a1, a2, a3 = p1_loop

        s0, s1 = jnp.sort(a0), jnp.sort(a1)
        s2, s3 = jnp.sort(a2), jnp.sort(a3)
        r1, r3 = _rev(s1), _rev(s3)
        p0 = jnp.sort(jnp.minimum(s0, r1))
        p1 = jnp.sort(jnp.maximum(s0, r1))
        q0 = jnp.sort(jnp.minimum(s2, r3))
        q1 = jnp.sort(jnp.maximum(s2, r3))
        lo0 = jnp.minimum(p0, _rev(q1))
        lo1 = jnp.minimum(p1, _rev(q0))
        t = jnp.max(jnp.maximum(lo0, lo1))

        @plsc.parallel_loop(0, NV // 2, unroll=2,
                            carry=(jnp.int32(0), jnp.int32(0)))
        def p2_loop(i, offs):
            off_a, off_b = offs
            o16 = i * L
            d = distv[pl.ds(o16, L)]
            sel = d <= t
            cnt = plsc.all_reduce_population_count(sel)[0]
            plsc.store_compressed(cand_a.at[pl.ds(off_a, L)], iota + o16,
                                  mask=sel)
            o16b = o16 + (N // 2)
            db = distv[pl.ds(o16b, L)]
            selb = db <= t
            cntb = plsc.all_reduce_population_count(selb)[0]
            plsc.store_compressed(cand_b.at[pl.ds(off_b, L)], iota + o16b,
                                  mask=selb)
            return (off_a + cnt, off_b + cntb)

        ca, cb = p2_loop

        def make_scan(cand, cnt):
            def scan_body(v, st):
                a0k, a0v, a1k, a1v = st
                o16 = v * L
                ii = cand[pl.ds(o16, L)]
                valid = (iota + o16) < cnt
                d = plsc.load_gather(distv, [ii], mask=valid)
                d = jnp.where(valid, d, F32_INF)
                ii = jnp.where(valid, ii, I32_MAX)
                sk, sv = plsc.sort_key_val(d, ii)
                mk, mv, _, _ = _lexminmax(a1k, a1v, _rev(sk), _rev(sv))
                mk, mv = plsc.sort_key_val(mk, mv)
                lok, lov, hik, hiv = _lexminmax(a0k, a0v, _rev(mk), _rev(mv))
                a0k, a0v = plsc.sort_key_val(lok, lov)
                a1k, a1v = plsc.sort_key_val(hik, hiv)
                return (a0k, a0v, a1k, a1v)
            return scan_body

        st = lax.fori_loop(0, (ca + L - 1) // L, make_scan(cand_a, ca),
                           (inf16, imax16, inf16, imax16))
        st = lax.fori_loop(0, (cb + L - 1) // L, make_scan(cand_b, cb), st)
        _, ov0, _, ov1 = st
        outbuf[pl.ds(ci * K, L)] = ov0
        outbuf[pl.ds(ci * K + L, L)] = ov1
        return 0

    lax.fori_loop(0, CPT, center_body, 0)
    pltpu.sync_copy(outbuf, out.at[pl.ds(wid * (CPT * K), CPT * K)])


@jax.jit
def _knn(xt, ct):
    f = pl.kernel(
        _knn_body,
        out_type=jax.ShapeDtypeStruct((B * NPOINT * K,), jnp.int32),
        mesh=plsc.VectorSubcoreMesh(core_axis_name="c", subcore_axis_name="s"),
        compiler_params=pltpu.CompilerParams(needs_layout_passes=False),
        scratch_types=[
            pltpu.VMEM((N,), jnp.float32),
            pltpu.VMEM((N,), jnp.float32),
            pltpu.VMEM((N,), jnp.float32),
            pltpu.VMEM((CPT * L,), jnp.float32),
            pltpu.VMEM((CPT * L,), jnp.float32),
            pltpu.VMEM((CPT * L,), jnp.float32),
            pltpu.VMEM((N,), jnp.float32),
            pltpu.VMEM((CAP // 2,), jnp.int32),
            pltpu.VMEM((CAP // 2,), jnp.int32),
            pltpu.VMEM((CPT * K,), jnp.int32),
        ],
    )
    return f(xt, ct)


def kernel(xyz, center):
    xt = jnp.transpose(xyz, (0, 2, 1)).reshape(B * 3 * N)
    ct = jnp.repeat(jnp.transpose(center, (0, 2, 1)).reshape(B * 3 * NPOINT), L)
    return _knn(xt, ct).reshape(B, NPOINT, K)


# speedup vs baseline: 1.3044x; 1.0651x over previous
import numpy as np

import jax
import jax.numpy as jnp
from jax import lax
from jax.experimental import pallas as pl
from jax.experimental.pallas import tpu as pltpu
from jax.experimental.pallas import tpu_sc as plsc

B = 4
NPOINT = 1024
N = 16384
K = 32
L = 16
NV = N // L
NTEC = 32
CPT = (B * NPOINT) // NTEC
TPB = NTEC // B
CAP = N + 2 * L

F32_INF = np.float32(np.inf)
F32_NINF = np.float32(-np.inf)
I32_MAX = np.int32(2**31 - 1)


def _rev(x):
    return lax.rev(x, (0,))


def _lexminmax(ak, av, bk, bv):
    m = (ak < bk) | ((ak == bk) & (av < bv))
    return (jnp.where(m, ak, bk), jnp.where(m, av, bv),
            jnp.where(m, bk, ak), jnp.where(m, bv, av))


def _knn_body(xt, ct, out, xv, yv, zv, cxv, cyv, czv, distv, cand_i,
              outbuf):
    wid = lax.axis_index("s") * 2 + lax.axis_index("c")
    b = wid // TPB
    c0 = (wid % TPB) * CPT

    pltpu.sync_copy(xt.at[pl.ds((b * 3 + 0) * N, N)], xv)
    pltpu.sync_copy(xt.at[pl.ds((b * 3 + 1) * N, N)], yv)
    pltpu.sync_copy(xt.at[pl.ds((b * 3 + 2) * N, N)], zv)
    pltpu.sync_copy(ct.at[pl.ds(((b * 3 + 0) * NPOINT + c0) * L, CPT * L)], cxv)
    pltpu.sync_copy(ct.at[pl.ds(((b * 3 + 1) * NPOINT + c0) * L, CPT * L)], cyv)
    pltpu.sync_copy(ct.at[pl.ds(((b * 3 + 2) * NPOINT + c0) * L, CPT * L)], czv)

    iota = lax.iota(jnp.int32, L)
    inf16 = jnp.full((L,), F32_INF, jnp.float32)
    imax16 = jnp.full((L,), I32_MAX, jnp.int32)

    def center_body(ci, _):
        cx = cxv[pl.ds(ci * L, L)]
        cy = cyv[pl.ds(ci * L, L)]
        cz = czv[pl.ds(ci * L, L)]

        @plsc.parallel_loop(0, NV, step=4, unroll=2,
                            carry=(inf16, inf16, inf16, inf16))
        def p1_loop(j, accs):
            new = []
            for r, a in enumerate(accs):
                off = (j + r) * L
                x = xv[pl.ds(off, L)]
                y = yv[pl.ds(off, L)]
                z = zv[pl.ds(off, L)]
                dx = cx - x
                dy = cy - y
                dz = cz - z
                d = (dx * dx + dy * dy) + dz * dz
                distv[pl.ds(off, L)] = d
                new.append(jnp.minimum(a, d))
            return tuple(new)

        a0, a1, a2, a3 = p1_loop

        s0, s1 = jnp.sort(a0), jnp.sort(a1)
        s2, s3 = jnp.sort(a2), jnp.sort(a3)
        r1, r3 = _rev(s1), _rev(s3)
        p0 = jnp.sort(jnp.minimum(s0, r1))
        p1 = jnp.sort(jnp.maximum(s0, r1))
        q0 = jnp.sort(jnp.minimum(s2, r3))
        q1 = jnp.sort(jnp.maximum(s2, r3))
        lo0 = jnp.minimum(p0, _rev(q1))
        lo1 = jnp.minimum(p1, _rev(q0))
        t = jnp.max(jnp.maximum(lo0, lo1))

        @plsc.parallel_loop(0, NV, unroll=4, carry=jnp.int32(0))
        def p2_loop(i, off):
            o16 = i * L
            d = distv[pl.ds(o16, L)]
            sel = d <= t
            cnt = plsc.all_reduce_population_count(sel)[0]
            plsc.store_compressed(cand_i.at[pl.ds(off, L)], iota + o16,
                                  mask=sel)
            return off + cnt

        c = p2_loop
        nv = (c + L - 1) // L

        def scan_body(v, st):
            a0k, a0v, a1k, a1v = st
            o16 = v * L
            ii = cand_i[pl.ds(o16, L)]
            valid = (iota + o16) < c
            d = plsc.load_gather(distv, [ii], mask=valid)
            d = jnp.where(valid, d, F32_INF)
            ii = jnp.where(valid, ii, I32_MAX)
            sk, sv = plsc.sort_key_val(d, ii)
            mk, mv, _, _ = _lexminmax(a1k, a1v, _rev(sk), _rev(sv))
            mk, mv = plsc.sort_key_val(mk, mv)
            lok, lov, hik, hiv = _lexminmax(a0k, a0v, _rev(mk), _rev(mv))
            a0k, a0v = plsc.sort_key_val(lok, lov)
            a1k, a1v = plsc.sort_key_val(hik, hiv)
            return (a0k, a0v, a1k, a1v)

        _, ov0, _, ov1 = lax.fori_loop(0, nv, scan_body,
                                       (inf16, imax16, inf16, imax16))
        outbuf[pl.ds(ci * K, L)] = ov0
        outbuf[pl.ds(ci * K + L, L)] = ov1
        return 0

    lax.fori_loop(0, CPT, center_body, 0)
    pltpu.sync_copy(outbuf, out.at[pl.ds(wid * (CPT * K), CPT * K)])


@jax.jit
def _knn(xt, ct):
    f = pl.kernel(
        _knn_body,
        out_type=jax.ShapeDtypeStruct((B * NPOINT * K,), jnp.int32),
        mesh=plsc.VectorSubcoreMesh(core_axis_name="c", subcore_axis_name="s"),
        compiler_params=pltpu.CompilerParams(needs_layout_passes=False),
        scratch_types=[
            pltpu.VMEM((N,), jnp.float32),
            pltpu.VMEM((N,), jnp.float32),
            pltpu.VMEM((N,), jnp.float32),
            pltpu.VMEM((CPT * L,), jnp.float32),
            pltpu.VMEM((CPT * L,), jnp.float32),
            pltpu.VMEM((CPT * L,), jnp.float32),
            pltpu.VMEM((N,), jnp.float32),
            pltpu.VMEM((CAP,), jnp.int32),
            pltpu.VMEM((CPT * K,), jnp.int32),
        ],
    )
    return f(xt, ct)


def kernel(xyz, center):
    xt = jnp.transpose(xyz, (0, 2, 1)).reshape(B * 3 * N)
    ct = jnp.repeat(jnp.transpose(center, (0, 2, 1)).reshape(B * 3 * NPOINT), L)
    return _knn(xt, ct).reshape(B, NPOINT, K)


# speedup vs baseline: 1.3060x; 1.0012x over previous
import numpy as np

import jax
import jax.numpy as jnp
from jax import lax
from jax.experimental import pallas as pl
from jax.experimental.pallas import tpu as pltpu
from jax.experimental.pallas import tpu_sc as plsc

B = 4
NPOINT = 1024
N = 16384
K = 32
L = 16
NV = N // L
NTEC = 32
CPT = (B * NPOINT) // NTEC
TPB = NTEC // B
CAP = N + 2 * L

F32_INF = np.float32(np.inf)
F32_NINF = np.float32(-np.inf)
I32_MAX = np.int32(2**31 - 1)


def _rev(x):
    return lax.rev(x, (0,))


def _lexminmax(ak, av, bk, bv):
    m = (ak < bk) | ((ak == bk) & (av < bv))
    return (jnp.where(m, ak, bk), jnp.where(m, av, bv),
            jnp.where(m, bk, ak), jnp.where(m, bv, av))


def _knn_body(xt, ct, out, xv, yv, zv, cxv, cyv, czv, distv, cand_i,
              outbuf):
    wid = lax.axis_index("s") * 2 + lax.axis_index("c")
    b = wid // TPB
    c0 = (wid % TPB) * CPT

    pltpu.sync_copy(xt.at[pl.ds((b * 3 + 0) * N, N)], xv)
    pltpu.sync_copy(xt.at[pl.ds((b * 3 + 1) * N, N)], yv)
    pltpu.sync_copy(xt.at[pl.ds((b * 3 + 2) * N, N)], zv)
    pltpu.sync_copy(ct.at[pl.ds(((b * 3 + 0) * NPOINT + c0) * L, CPT * L)], cxv)
    pltpu.sync_copy(ct.at[pl.ds(((b * 3 + 1) * NPOINT + c0) * L, CPT * L)], cyv)
    pltpu.sync_copy(ct.at[pl.ds(((b * 3 + 2) * NPOINT + c0) * L, CPT * L)], czv)

    iota = lax.iota(jnp.int32, L)
    inf16 = jnp.full((L,), F32_INF, jnp.float32)
    imax16 = jnp.full((L,), I32_MAX, jnp.int32)

    def center_body(ci, _):
        cx = cxv[pl.ds(ci * L, L)]
        cy = cyv[pl.ds(ci * L, L)]
        cz = czv[pl.ds(ci * L, L)]

        @plsc.parallel_loop(0, NV, step=4, unroll=4,
                            carry=(inf16, inf16, inf16, inf16))
        def p1_loop(j, accs):
            new = []
            for r, a in enumerate(accs):
                off = (j + r) * L
                x = xv[pl.ds(off, L)]
                y = yv[pl.ds(off, L)]
                z = zv[pl.ds(off, L)]
                dx = cx - x
                dy = cy - y
                dz = cz - z
                d = (dx * dx + dy * dy) + dz * dz
                distv[pl.ds(off, L)] = d
                new.append(jnp.minimum(a, d))
            return tuple(new)

        a0, a1, a2, a3 = p1_loop

        s0, s1 = jnp.sort(a0), jnp.sort(a1)
        s2, s3 = jnp.sort(a2), jnp.sort(a3)
        r1, r3 = _rev(s1), _rev(s3)
        p0 = jnp.sort(jnp.minimum(s0, r1))
        p1 = jnp.sort(jnp.maximum(s0, r1))
        q0 = jnp.sort(jnp.minimum(s2, r3))
        q1 = jnp.sort(jnp.maximum(s2, r3))
        lo0 = jnp.minimum(p0, _rev(q1))
        lo1 = jnp.minimum(p1, _rev(q0))
        t = jnp.max(jnp.maximum(lo0, lo1))

        @plsc.parallel_loop(0, NV, unroll=4, carry=jnp.int32(0))
        def p2_loop(i, off):
            o16 = i * L
            d = distv[pl.ds(o16, L)]
            sel = d <= t
            cnt = plsc.all_reduce_population_count(sel)[0]
            plsc.store_compressed(cand_i.at[pl.ds(off, L)], iota + o16,
                                  mask=sel)
            return off + cnt

        c = p2_loop
        nv = (c + L - 1) // L

        def scan_body(v, st):
            a0k, a0v, a1k, a1v = st
            o16 = v * L
            ii = cand_i[pl.ds(o16, L)]
            valid = (iota + o16) < c
            d = plsc.load_gather(distv, [ii], mask=valid)
            d = jnp.where(valid, d, F32_INF)
            ii = jnp.where(valid, ii, I32_MAX)
            sk, sv = plsc.sort_key_val(d, ii)
            mk, mv, _, _ = _lexminmax(a1k, a1v, _rev(sk), _rev(sv))
            mk, mv = plsc.sort_key_val(mk, mv)
            lok, lov, hik, hiv = _lexminmax(a0k, a0v, _rev(mk), _rev(mv))
            a0k, a0v = plsc.sort_key_val(lok, lov)
            a1k, a1v = plsc.sort_key_val(hik, hiv)
            return (a0k, a0v, a1k, a1v)

        _, ov0, _, ov1 = lax.fori_loop(0, nv, scan_body,
                                       (inf16, imax16, inf16, imax16))
        outbuf[pl.ds(ci * K, L)] = ov0
        outbuf[pl.ds(ci * K + L, L)] = ov1
        return 0

    lax.fori_loop(0, CPT, center_body, 0)
    pltpu.sync_copy(outbuf, out.at[pl.ds(wid * (CPT * K), CPT * K)])


@jax.jit
def _knn(xt, ct):
    f = pl.kernel(
        _knn_body,
        out_type=jax.ShapeDtypeStruct((B * NPOINT * K,), jnp.int32),
        mesh=plsc.VectorSubcoreMesh(core_axis_name="c", subcore_axis_name="s"),
        compiler_params=pltpu.CompilerParams(needs_layout_passes=False),
        scratch_types=[
            pltpu.VMEM((N,), jnp.float32),
            pltpu.VMEM((N,), jnp.float32),
            pltpu.VMEM((N,), jnp.float32),
            pltpu.VMEM((CPT * L,), jnp.float32),
            pltpu.VMEM((CPT * L,), jnp.float32),
            pltpu.VMEM((CPT * L,), jnp.float32),
            pltpu.VMEM((N,), jnp.float32),
            pltpu.VMEM((CAP,), jnp.int32),
            pltpu.VMEM((CPT * K,), jnp.int32),
        ],
    )
    return f(xt, ct)


def kernel(xyz, center):
    xt = jnp.transpose(xyz, (0, 2, 1)).reshape(B * 3 * N)
    ct = jnp.repeat(jnp.transpose(center, (0, 2, 1)).reshape(B * 3 * NPOINT), L)
    return _knn(xt, ct).reshape(B, NPOINT, K)
